# Initial kernel scaffold; baseline (speedup 1.0000x reference)
import jax
import jax.numpy as jnp
from jax.experimental import pallas as pl


def kernel(adv_pc, ori_pc):
    raise NotImplementedError("write your pallas kernel here")


# speedup vs baseline: 16.7945x; 16.7945x over previous
import functools

import jax
import jax.numpy as jnp
from jax.experimental import pallas as pl

_N = 1024
_KNN_K = 5
_ALPHA = 1.05
_W1 = 5.0
_W2 = 3.0
_BIG = 3.0e38


def _body(a_ref, at_ref, ot_ref, l1_ref, knn_ref):
    a = a_ref[0]
    at = at_ref[0]
    ot = ot_ref[0]

    aa_col = jnp.sum(a * a, axis=1, keepdims=True)
    aa_row = jnp.sum(at * at, axis=0, keepdims=True)
    oo_row = jnp.sum(ot * ot, axis=0, keepdims=True)

    inner1 = jnp.dot(a, ot, preferred_element_type=jnp.float32)
    d1 = aa_col + (-2.0) * inner1 + oo_row
    l1 = jnp.mean(jnp.min(d1, axis=1))
    l1_ref[...] = jnp.full((1, 1, 128), l1, jnp.float32)

    inner2 = jnp.dot(a, at, preferred_element_type=jnp.float32)
    d2 = aa_col + (-2.0) * inner2 + aa_row

    col = jax.lax.broadcasted_iota(jnp.int32, (_N, _N), 1)
    d = d2
    total = jnp.zeros((_N, 1), jnp.float32)
    smallest = None
    for j in range(_KNN_K + 1):
        m = jnp.min(d, axis=1, keepdims=True)
        total = total + m
        if j == 0:
            smallest = m
        if j < _KNN_K:
            idx = jnp.min(jnp.where(d == m, col, _N), axis=1, keepdims=True)
            d = jnp.where(col == idx, _BIG, d)

    value = (total - smallest) * (1.0 / _KNN_K)
    mean = jnp.mean(value)
    std = jnp.sqrt(jnp.sum((value - mean) ** 2) * (1.0 / (_N - 1)))
    thr = mean + _ALPHA * std
    w = (value > thr).astype(jnp.float32)
    knn = jnp.mean(value * w)
    knn_ref[...] = jnp.full((1, 1, 128), knn, jnp.float32)


@functools.partial(jax.jit, static_argnames=())
def kernel(adv_pc, ori_pc):
    B = adv_pc.shape[0]
    pad = ((0, 0), (0, 0), (0, 5))
    a = jnp.pad(adv_pc, pad)
    at = a.transpose(0, 2, 1)
    ot = jnp.pad(ori_pc, pad).transpose(0, 2, 1)

    l1, knn = pl.pallas_call(
        _body,
        grid=(B,),
        in_specs=[
            pl.BlockSpec((1, _N, 8), lambda b: (b, 0, 0)),
            pl.BlockSpec((1, 8, _N), lambda b: (b, 0, 0)),
            pl.BlockSpec((1, 8, _N), lambda b: (b, 0, 0)),
        ],
        out_specs=[
            pl.BlockSpec((1, 1, 128), lambda b: (b, 0, 0)),
            pl.BlockSpec((1, 1, 128), lambda b: (b, 0, 0)),
        ],
        out_shape=[
            jax.ShapeDtypeStruct((B, 1, 128), jnp.float32),
            jax.ShapeDtypeStruct((B, 1, 128), jnp.float32),
        ],
    )(a, at, ot)

    chamfer_loss = jnp.mean(l1[:, 0, 0])
    knn_loss = jnp.mean(knn[:, 0, 0])
    return chamfer_loss * _W1 + knn_loss * _W2


# speedup vs baseline: 31.2033x; 1.8580x over previous
import functools

import jax
import jax.numpy as jnp
from jax.experimental import pallas as pl

_N = 1024
_KNN_K = 5
_ALPHA = 1.05
_W1 = 5.0
_W2 = 3.0
_BIG = 3.0e38


def _body(m2_ref, m1_ref, a2t_ref, l1_ref, knn_ref):
    m2 = m2_ref[0]
    m1 = m1_ref[0]
    a2t = a2t_ref[0]

    d2 = jnp.dot(m2, a2t, preferred_element_type=jnp.float32)
    d1t = jnp.dot(m1, a2t, preferred_element_type=jnp.float32)

    cm = d1t[0:8, :]
    for k in range(1, _N // 8):
        cm = jnp.minimum(cm, d1t[k * 8:(k + 1) * 8, :])
    l1 = jnp.mean(jnp.min(cm, axis=0))
    l1_ref[...] = jnp.full((1, 1, 128), l1, jnp.float32)

    R = [jnp.full((8, _N), _BIG, jnp.float32) for _ in range(6)]
    for k in range(_N // 8):
        x = d2[k * 8:(k + 1) * 8, :]
        for j in range(5):
            mj = jnp.minimum(R[j], x)
            x = jnp.maximum(R[j], x)
            R[j] = mj
        R[5] = jnp.minimum(R[5], x)

    S = jnp.concatenate(R, axis=0)
    row = jax.lax.broadcasted_iota(jnp.int32, (48, _N), 0)
    acc = jnp.zeros((1, _N), jnp.float32)
    smallest = None
    for j in range(_KNN_K + 1):
        m = jnp.min(S, axis=0, keepdims=True)
        acc = acc + m
        if j == 0:
            smallest = m
        if j < _KNN_K:
            idx = jnp.min(jnp.where(S == m, row, 48), axis=0, keepdims=True)
            S = jnp.where(row == idx, _BIG, S)

    value = (acc - smallest) * (1.0 / _KNN_K)
    mean = jnp.mean(value)
    std = jnp.sqrt(jnp.sum((value - mean) ** 2) * (1.0 / (_N - 1)))
    thr = mean + _ALPHA * std
    w = (value > thr).astype(jnp.float32)
    knn = jnp.mean(value * w)
    knn_ref[...] = jnp.full((1, 1, 128), knn, jnp.float32)


@functools.partial(jax.jit, static_argnames=())
def kernel(adv_pc, ori_pc):
    B = adv_pc.shape[0]
    aa = jnp.sum(adv_pc * adv_pc, axis=-1, keepdims=True)
    oo = jnp.sum(ori_pc * ori_pc, axis=-1, keepdims=True)
    ones = jnp.ones_like(aa)
    zeros = jnp.zeros_like(adv_pc)
    m2 = jnp.concatenate([-2.0 * adv_pc, aa, ones, zeros], axis=-1)
    m1 = jnp.concatenate([-2.0 * ori_pc, oo, ones, zeros], axis=-1)
    a2 = jnp.concatenate([adv_pc, ones, aa, zeros], axis=-1)
    a2t = a2.transpose(0, 2, 1)

    l1, knn = pl.pallas_call(
        _body,
        grid=(B,),
        in_specs=[
            pl.BlockSpec((1, _N, 8), lambda b: (b, 0, 0)),
            pl.BlockSpec((1, _N, 8), lambda b: (b, 0, 0)),
            pl.BlockSpec((1, 8, _N), lambda b: (b, 0, 0)),
        ],
        out_specs=[
            pl.BlockSpec((1, 1, 128), lambda b: (b, 0, 0)),
            pl.BlockSpec((1, 1, 128), lambda b: (b, 0, 0)),
        ],
        out_shape=[
            jax.ShapeDtypeStruct((B, 1, 128), jnp.float32),
            jax.ShapeDtypeStruct((B, 1, 128), jnp.float32),
        ],
    )(m2, m1, a2t)

    chamfer_loss = jnp.mean(l1[:, 0, 0])
    knn_loss = jnp.mean(knn[:, 0, 0])
    return chamfer_loss * _W1 + knn_loss * _W2
